# baseline (speedup 1.0000x reference)
import jax
import jax.numpy as jnp
from jax.experimental import pallas as pl
from jax.experimental.pallas import tpu as pltpu


def _body(td_ref, lae_ref, ll_ref, mask_ref, ltm_ref, ev_ref, ne_ref, cnt_ref, *, inv_n):
    i = pl.program_id(0)

    @pl.when(i == 0)
    def _init():
        ev_ref[0, 0] = jnp.float32(0.0)
        ne_ref[0, 0] = jnp.float32(0.0)
        cnt_ref[0, 0] = jnp.int32(0)

    maskf = mask_ref[...]
    w = td_ref[...] * maskf * inv_n
    ll = ll_ref[...]
    ne_ref[0, 0] += jnp.sum(ll * w[:, :, None])

    ev_lam = jnp.sum(lae_ref[...] * ltm_ref[...], axis=2)
    ev_ref[0, 0] += jnp.sum(jnp.log(jnp.where(maskf > 0, ev_lam, 1.0)))
    cnt_ref[0, 0] += jnp.sum(maskf).astype(jnp.int32)


def kernel(time_delta_seq, lambda_at_event, lambdas_loss_samples, seq_mask, lambda_type_mask):
    B, S, N, K = lambdas_loss_samples.shape
    ll = lambdas_loss_samples.reshape(B, S, N * K)
    maskf = seq_mask.astype(jnp.float32)

    CH = 128
    grid = (S // CH,)

    import functools
    body = functools.partial(_body, inv_n=1.0 / N)
    ev, ne, cnt = pl.pallas_call(
        body,
        grid=grid,
        in_specs=[
            pl.BlockSpec((B, CH), lambda i: (0, i)),
            pl.BlockSpec((B, CH, K), lambda i: (0, i, 0)),
            pl.BlockSpec((B, CH, N * K), lambda i: (0, i, 0)),
            pl.BlockSpec((B, CH), lambda i: (0, i)),
            pl.BlockSpec((B, CH, K), lambda i: (0, i, 0)),
        ],
        out_specs=[
            pl.BlockSpec(memory_space=pltpu.SMEM),
            pl.BlockSpec(memory_space=pltpu.SMEM),
            pl.BlockSpec(memory_space=pltpu.SMEM),
        ],
        out_shape=[
            jax.ShapeDtypeStruct((1, 1), jnp.float32),
            jax.ShapeDtypeStruct((1, 1), jnp.float32),
            jax.ShapeDtypeStruct((1, 1), jnp.int32),
        ],
    )(time_delta_seq, lambda_at_event, ll, maskf, lambda_type_mask)

    return (ev[0, 0], ne[0, 0], cnt[0, 0])
